# baseline (speedup 1.0000x reference)
import jax
import jax.numpy as jnp
from jax import lax
from jax.experimental import pallas as pl
from jax.experimental.pallas import tpu as pltpu
from jax.experimental.pallas import tpu_sc as plsc

B, H, W_ = 16, 224, 224
D = 64
HW = H * W_
N = B * HW
NW = 32
PER_W = N // NW
C = 256
NCHUNK = PER_W // C
IDX_L = 128
IDX_ROWS = PER_W // IDX_L
GPC = C // IDX_L
IDX_PAD = 4
PITCH = 264
NBUF = 2


def _body(x_hbm, tbl_hbm, out_hbm, idx_v, rows_v, tb0, tb1,
          g0, g1, g2, g3, o0, o1):
    cid = lax.axis_index("c")
    sid = lax.axis_index("s")
    w = sid * 2 + cid
    b = w // 2
    half = w % 2
    gsems = [g0, g1, g2, g3][:NBUF]

    shift = IDX_PAD * half
    pltpu.sync_copy(
        x_hbm.at[pl.ds(w * IDX_ROWS - shift, IDX_ROWS + IDX_PAD), :], idx_v)

    iota = lax.iota(jnp.int32, 16)
    d_base = [(iota + 16 * q) * PITCH for q in range(4)]

    def issue_gather(cidx, buf):
        for sub in range(GPC):
            pltpu.async_copy(
                tbl_hbm.at[idx_v.at[shift + GPC * cidx + sub]],
                rows_v.at[buf, pl.ds(sub * IDX_L, IDX_L), :],
                gsems[buf],
            )

    def wait_gather(buf):
        for sub in range(GPC):
            pltpu.make_async_copy(
                tbl_hbm.at[idx_v.at[0]],
                rows_v.at[buf, pl.ds(sub * IDX_L, IDX_L), :],
                gsems[buf],
            ).wait()

    def wait_out(osem):
        pltpu.make_async_copy(
            out_hbm.at[0, pl.ds(0, D), pl.ds(0, C)],
            out_hbm.at[0, pl.ds(0, D), pl.ds(0, C)],
            osem,
        ).wait()

    for c0 in range(NBUF - 1):
        issue_gather(c0, c0)

    def chunk_step(cidx, buf, par, tail):
        osem = o0 if par == 0 else o1
        tb = tb0 if par == 0 else tb1

        if not tail:
            @pl.when(cidx + NBUF - 1 < NCHUNK)
            def _():
                issue_gather(cidx + NBUF - 1, (buf + NBUF - 1) % NBUF)

        wait_gather(buf)

        if tail:
            wait_out(osem)
        else:
            @pl.when(cidx >= 2)
            def _():
                wait_out(osem)

        @plsc.parallel_loop(0, C, unroll=16)
        def tr(j):
            jv = jnp.full((16,), j, jnp.int32)
            for q in range(4):
                v = rows_v[buf, j, pl.ds(16 * q, 16)]
                plsc.store_scatter(tb, [d_base[q] + jv], v)

        col = (half * NCHUNK + cidx) * C
        for d in range(D):
            pltpu.async_copy(
                tb.at[pl.ds(d * PITCH, C)],
                out_hbm.at[b, d, pl.ds(col, C)],
                osem,
            )

    def outer(ii, carry):
        for p in range(NBUF):
            chunk_step(NBUF * ii + p, p, p % 2, False)
        return carry

    lax.fori_loop(0, NCHUNK // NBUF, outer, None)
    for k in range((NCHUNK // NBUF) * NBUF, NCHUNK):
        chunk_step(k, k % NBUF, k % 2, True)

    wait_out(o0)
    wait_out(o1)

    wait_out(o0)
    wait_out(o1)


@jax.jit
def _run(x2, table):
    mesh = plsc.VectorSubcoreMesh(core_axis_name="c", subcore_axis_name="s")
    f = pl.kernel(
        _body,
        out_type=jax.ShapeDtypeStruct((B, D, HW), jnp.float32),
        mesh=mesh,
        compiler_params=pltpu.CompilerParams(use_tc_tiling_on_sc=False, needs_layout_passes=False),
        scratch_types=[
            pltpu.VMEM((IDX_ROWS + IDX_PAD, IDX_L), jnp.int32),
            pltpu.VMEM((NBUF, C, D), jnp.float32),
            pltpu.VMEM((D * PITCH,), jnp.float32),
            pltpu.VMEM((D * PITCH,), jnp.float32),
            pltpu.SemaphoreType.DMA,
            pltpu.SemaphoreType.DMA,
            pltpu.SemaphoreType.DMA,
            pltpu.SemaphoreType.DMA,
            pltpu.SemaphoreType.DMA,
            pltpu.SemaphoreType.DMA,
        ],
    )
    return f(x2, table)


def kernel(x, table):
    x2 = x.reshape(N // IDX_L, IDX_L).astype(jnp.int32)
    out = _run(x2, table)
    return out.reshape(B, D, H, W_)


# speedup vs baseline: 1.0016x; 1.0016x over previous
import jax
import jax.numpy as jnp
from jax import lax
from jax.experimental import pallas as pl
from jax.experimental.pallas import tpu as pltpu
from jax.experimental.pallas import tpu_sc as plsc

B, H, W_ = 16, 224, 224
D = 64
HW = H * W_
N = B * HW
NW = 32
PER_W = N // NW
C = 256
NCHUNK = PER_W // C
IDX_L = 128
IDX_ROWS = PER_W // IDX_L
GPC = C // IDX_L
IDX_PAD = 4
PITCH = 264
NBUF = 2


def _body(x_hbm, tbl_hbm, out_hbm, idx_v, rows_v, tb0, tb1,
          g0, g1, g2, g3, o0, o1):
    cid = lax.axis_index("c")
    sid = lax.axis_index("s")
    w = sid * 2 + cid
    b = w // 2
    half = w % 2
    gsems = [g0, g1, g2, g3][:NBUF]

    shift = IDX_PAD * half
    pltpu.sync_copy(
        x_hbm.at[pl.ds(w * IDX_ROWS - shift, IDX_ROWS + IDX_PAD), :], idx_v)

    iota = lax.iota(jnp.int32, 16)
    d_base = [(iota + 16 * q) * PITCH for q in range(4)]

    def issue_gather(cidx, buf):
        for sub in range(GPC):
            pltpu.async_copy(
                tbl_hbm.at[idx_v.at[shift + GPC * cidx + sub]],
                rows_v.at[buf, pl.ds(sub * IDX_L, IDX_L), :],
                gsems[buf],
            )

    def wait_gather(buf):
        for sub in range(GPC):
            pltpu.make_async_copy(
                tbl_hbm.at[idx_v.at[0]],
                rows_v.at[buf, pl.ds(sub * IDX_L, IDX_L), :],
                gsems[buf],
            ).wait()

    def wait_out(osem):
        pltpu.make_async_copy(
            out_hbm.at[0, pl.ds(0, D), pl.ds(0, C)],
            out_hbm.at[0, pl.ds(0, D), pl.ds(0, C)],
            osem,
        ).wait()

    for c0 in range(NBUF - 1):
        issue_gather(c0, c0)

    def chunk_step(cidx, buf, par, tail):
        osem = o0 if par == 0 else o1
        tb = tb0 if par == 0 else tb1

        if not tail:
            @pl.when(cidx + NBUF - 1 < NCHUNK)
            def _():
                issue_gather(cidx + NBUF - 1, (buf + NBUF - 1) % NBUF)

        wait_gather(buf)

        if tail:
            wait_out(osem)
        else:
            @pl.when(cidx >= 2)
            def _():
                wait_out(osem)

        @plsc.parallel_loop(0, C, unroll=16)
        def tr(j):
            jv = jnp.full((16,), j, jnp.int32)
            for q in range(4):
                v = rows_v[buf, j, pl.ds(16 * q, 16)]
                plsc.store_scatter(tb, [d_base[q] + jv], v)

        col = (half * NCHUNK + cidx) * C
        for d in range(D):
            pltpu.async_copy(
                tb.at[pl.ds(d * PITCH, C)],
                out_hbm.at[b, d, pl.ds(col, C)],
                osem,
            )

    def outer(ii, carry):
        for p in range(NBUF):
            chunk_step(NBUF * ii + p, p, p % 2, False)
        return carry

    lax.fori_loop(0, NCHUNK // NBUF, outer, None)
    for k in range((NCHUNK // NBUF) * NBUF, NCHUNK):
        chunk_step(k, k % NBUF, k % 2, True)

    wait_out(o0)
    wait_out(o1)


@jax.jit
def _run(x2, table):
    mesh = plsc.VectorSubcoreMesh(core_axis_name="c", subcore_axis_name="s")
    f = pl.kernel(
        _body,
        out_type=jax.ShapeDtypeStruct((B, D, HW), jnp.float32),
        mesh=mesh,
        compiler_params=pltpu.CompilerParams(use_tc_tiling_on_sc=False, needs_layout_passes=False),
        scratch_types=[
            pltpu.VMEM((IDX_ROWS + IDX_PAD, IDX_L), jnp.int32),
            pltpu.VMEM((NBUF, C, D), jnp.float32),
            pltpu.VMEM((D * PITCH,), jnp.float32),
            pltpu.VMEM((D * PITCH,), jnp.float32),
            pltpu.SemaphoreType.DMA,
            pltpu.SemaphoreType.DMA,
            pltpu.SemaphoreType.DMA,
            pltpu.SemaphoreType.DMA,
            pltpu.SemaphoreType.DMA,
            pltpu.SemaphoreType.DMA,
        ],
    )
    return f(x2, table)


def kernel(x, table):
    x2 = x.reshape(N // IDX_L, IDX_L).astype(jnp.int32)
    out = _run(x2, table)
    return out.reshape(B, D, H, W_)
